# Initial kernel scaffold; baseline (speedup 1.0000x reference)
import jax
import jax.numpy as jnp
from jax.experimental import pallas as pl


def kernel(loc, conf, dbox_list, targets):
    raise NotImplementedError("write your pallas kernel here")


# speedup vs baseline: 8.9124x; 8.9124x over previous
import jax
import jax.numpy as jnp
from jax.experimental import pallas as pl
from jax.experimental.pallas import tpu as pltpu
from functools import partial

_JACCARD_THRESH = 0.5
_NEGPOS_RATIO = 3
_VAR0 = 0.1
_VAR1 = 0.2
_NOBJ = 8


def _loss_kernel(tgt_ref, loc_ref, conf_ref, pf_ref, dbox_ref, out_ref):
    b = pl.program_id(0)
    D = loc_ref.shape[2]
    C = conf_ref.shape[1]

    pf0 = pf_ref[0:1, :]
    pf1 = pf_ref[1:2, :]
    pf2 = pf_ref[2:3, :]
    pf3 = pf_ref[3:4, :]
    area_b = (pf2 - pf0) * (pf3 - pf1)

    iota = jax.lax.broadcasted_iota(jnp.int32, (1, D), 1)

    ov = []
    tx = [[tgt_ref[0, i, j] for j in range(4)] for i in range(_NOBJ)]
    labels = [tgt_ref[0, i, 4] for i in range(_NOBJ)]
    for i in range(_NOBJ):
        x0, y0, x1, y1 = tx[i]
        iw = jnp.maximum(jnp.minimum(x1, pf2) - jnp.maximum(x0, pf0), 0.0)
        ih = jnp.maximum(jnp.minimum(y1, pf3) - jnp.maximum(y0, pf1), 0.0)
        inter = iw * ih
        area_a = (x1 - x0) * (y1 - y0)
        ov.append(inter / (area_a + area_b - inter))

    bto = ov[0]
    bti = jnp.zeros((1, D), jnp.int32)
    for i in range(1, _NOBJ):
        better = ov[i] > bto
        bto = jnp.where(better, ov[i], bto)
        bti = jnp.where(better, i, bti)

    big = jnp.int32(2 ** 30)
    for i in range(_NOBJ):
        m_i = jnp.max(ov[i])
        cand = jnp.where(ov[i] == m_i, iota, big)
        idx_i = jnp.min(cand)
        m = iota == idx_i
        bto = jnp.where(m, 2.0, bto)
        bti = jnp.where(m, i, bti)

    pos = bto >= _JACCARD_THRESH
    posf = pos.astype(jnp.float32)
    num_pos = jnp.sum(posf)

    mx0 = jnp.zeros((1, D), jnp.float32)
    mx1 = jnp.zeros((1, D), jnp.float32)
    mx2 = jnp.zeros((1, D), jnp.float32)
    mx3 = jnp.zeros((1, D), jnp.float32)
    lab = jnp.zeros((1, D), jnp.float32)
    for i in range(_NOBJ):
        sel = bti == i
        mx0 = jnp.where(sel, tx[i][0], mx0)
        mx1 = jnp.where(sel, tx[i][1], mx1)
        mx2 = jnp.where(sel, tx[i][2], mx2)
        mx3 = jnp.where(sel, tx[i][3], mx3)
        lab = jnp.where(sel, labels[i], lab)

    conf_t = jnp.where(pos, lab + 1.0, 0.0).astype(jnp.int32)

    dcx = dbox_ref[0:1, :]
    dcy = dbox_ref[1:2, :]
    dw = dbox_ref[2:3, :]
    dh = dbox_ref[3:4, :]
    g0 = ((mx0 + mx2) * 0.5 - dcx) / (_VAR0 * dw)
    g1 = ((mx1 + mx3) * 0.5 - dcy) / (_VAR0 * dh)
    w_ratio = jnp.where(pos, (mx2 - mx0) / dw, 1.0)
    h_ratio = jnp.where(pos, (mx3 - mx1) / dh, 1.0)
    g2 = jnp.log(w_ratio) / _VAR1
    g3 = jnp.log(h_ratio) / _VAR1

    loss_l = jnp.float32(0.0)
    for g, r in ((g0, 0), (g1, 1), (g2, 2), (g3, 3)):
        d = jnp.abs(loc_ref[0, r, :].reshape(1, D) - g)
        sl1 = jnp.where(d < 1.0, 0.5 * d * d, d - 0.5)
        loss_l = loss_l + jnp.sum(jnp.where(pos, sl1, 0.0))

    c = conf_ref[0]
    cmax = jnp.max(c, axis=0, keepdims=True)
    ssum = jnp.sum(jnp.exp(c - cmax), axis=0, keepdims=True)
    lse = cmax + jnp.log(ssum)
    picked = jnp.zeros((1, D), jnp.float32)
    for cc in range(C):
        picked = jnp.where(conf_t == cc, c[cc : cc + 1, :], picked)
    loss_c = lse - picked

    masked = jnp.where(pos, 0.0, loss_c)
    bits = jax.lax.bitcast_convert_type(masked, jnp.int32)
    k = jnp.minimum(num_pos.astype(jnp.int32) * _NEGPOS_RATIO, D)

    def body(_, lohi):
        lo, hi = lohi
        mid = lo + (hi - lo + 1) // 2
        cnt = jnp.sum((bits >= mid).astype(jnp.int32))
        ok = cnt >= k
        return jnp.where(ok, mid, lo), jnp.where(ok, hi, mid - 1)

    lo, _ = jax.lax.fori_loop(0, 31, body, (jnp.int32(0), jnp.int32(0x7F7FFFFF)))
    vk = jax.lax.bitcast_convert_type(lo, jnp.float32)
    gt = masked > vk
    cnt_gt = jnp.sum(gt.astype(jnp.float32))
    sum_gt = jnp.sum(jnp.where(gt, masked, 0.0))
    topk_sum = jnp.where(k > 0, sum_gt + (k.astype(jnp.float32) - cnt_gt) * vk, 0.0)
    loss_c_img = jnp.sum(jnp.where(pos, loss_c, 0.0)) + topk_sum

    @pl.when(b == 0)
    def _init():
        out_ref[0, 0] = loss_l
        out_ref[0, 1] = loss_c_img
        out_ref[0, 2] = num_pos

    @pl.when(b != 0)
    def _acc():
        out_ref[0, 0] += loss_l
        out_ref[0, 1] += loss_c_img
        out_ref[0, 2] += num_pos


@jax.jit
def kernel(loc, conf, dbox_list, targets):
    B, D, C = conf.shape
    loc_r = loc.transpose(0, 2, 1)
    conf_r = conf.transpose(0, 2, 1)
    pf = jnp.concatenate(
        [dbox_list[:, :2] - dbox_list[:, 2:] / 2.0,
         dbox_list[:, :2] + dbox_list[:, 2:] / 2.0], axis=1).T
    dbox_r = dbox_list.T

    out = pl.pallas_call(
        _loss_kernel,
        grid=(B,),
        in_specs=[
            pl.BlockSpec((1, _NOBJ, 5), lambda b: (b, 0, 0), memory_space=pltpu.SMEM),
            pl.BlockSpec((1, 4, D), lambda b: (b, 0, 0)),
            pl.BlockSpec((1, C, D), lambda b: (b, 0, 0)),
            pl.BlockSpec((4, D), lambda b: (0, 0)),
            pl.BlockSpec((4, D), lambda b: (0, 0)),
        ],
        out_specs=pl.BlockSpec((1, 3), lambda b: (0, 0), memory_space=pltpu.SMEM),
        out_shape=jax.ShapeDtypeStruct((1, 3), jnp.float32),
    )(targets, loc_r, conf_r, pf, dbox_r)

    N = out[0, 2]
    return (out[0, 0] / N, out[0, 1] / N)


# speedup vs baseline: 46.1699x; 5.1804x over previous
import jax
import jax.numpy as jnp
from jax.experimental import pallas as pl
from jax.experimental.pallas import tpu as pltpu

_JACCARD_THRESH = 0.5
_NEGPOS_RATIO = 3
_VAR0 = 0.1
_VAR1 = 0.2
_NOBJ = 8
_G = 8


def _loss_kernel(tgt_ref, loc_ref, conf_ref, pf_ref, dbox_ref, out_ref):
    b = pl.program_id(0)
    D = loc_ref.shape[3]
    C = conf_ref.shape[1]

    pf0 = pf_ref[0:1, :]
    pf1 = pf_ref[1:2, :]
    pf2 = pf_ref[2:3, :]
    pf3 = pf_ref[3:4, :]
    area_b = (pf2 - pf0) * (pf3 - pf1)

    iota = jax.lax.broadcasted_iota(jnp.int32, (1, D), 1)

    def tcol(i, j):
        return tgt_ref[0, :, i * 5 + j : i * 5 + j + 1]

    ov = []
    for i in range(_NOBJ):
        x0, y0, x1, y1 = tcol(i, 0), tcol(i, 1), tcol(i, 2), tcol(i, 3)
        iw = jnp.maximum(jnp.minimum(x1, pf2) - jnp.maximum(x0, pf0), 0.0)
        ih = jnp.maximum(jnp.minimum(y1, pf3) - jnp.maximum(y0, pf1), 0.0)
        inter = iw * ih
        area_a = (x1 - x0) * (y1 - y0)
        ov.append(inter / (area_a + area_b - inter))

    bto = ov[0]
    bti = jnp.zeros((_G, D), jnp.int32)
    for i in range(1, _NOBJ):
        better = ov[i] > bto
        bto = jnp.where(better, ov[i], bto)
        bti = jnp.where(better, i, bti)

    big = jnp.int32(2 ** 30)
    for i in range(_NOBJ):
        m_i = jnp.max(ov[i], axis=1, keepdims=True)
        cand = jnp.where(ov[i] == m_i, iota, big)
        idx_i = jnp.min(cand, axis=1, keepdims=True)
        m = iota == idx_i
        bto = jnp.where(m, 2.0, bto)
        bti = jnp.where(m, i, bti)

    pos = bto >= _JACCARD_THRESH
    posf = pos.astype(jnp.float32)
    num_pos = jnp.sum(posf, axis=1, keepdims=True)

    mx0 = jnp.zeros((_G, D), jnp.float32)
    mx1 = jnp.zeros((_G, D), jnp.float32)
    mx2 = jnp.zeros((_G, D), jnp.float32)
    mx3 = jnp.zeros((_G, D), jnp.float32)
    lab = jnp.zeros((_G, D), jnp.float32)
    for i in range(_NOBJ):
        sel = bti == i
        mx0 = jnp.where(sel, tcol(i, 0), mx0)
        mx1 = jnp.where(sel, tcol(i, 1), mx1)
        mx2 = jnp.where(sel, tcol(i, 2), mx2)
        mx3 = jnp.where(sel, tcol(i, 3), mx3)
        lab = jnp.where(sel, tcol(i, 4), lab)

    conf_t = jnp.where(pos, lab + 1.0, 0.0).astype(jnp.int32)

    dcx = dbox_ref[0:1, :]
    dcy = dbox_ref[1:2, :]
    dw = dbox_ref[2:3, :]
    dh = dbox_ref[3:4, :]
    g0 = ((mx0 + mx2) * 0.5 - dcx) / (_VAR0 * dw)
    g1 = ((mx1 + mx3) * 0.5 - dcy) / (_VAR0 * dh)
    g2 = jnp.log((mx2 - mx0) / dw) / _VAR1
    g3 = jnp.log((mx3 - mx1) / dh) / _VAR1

    loss_l = jnp.float32(0.0)
    for g, r in ((g0, 0), (g1, 1), (g2, 2), (g3, 3)):
        d = jnp.abs(loc_ref[0, r] - g)
        sl1 = jnp.where(d < 1.0, 0.5 * d * d, d - 0.5)
        loss_l = loss_l + jnp.sum(jnp.where(pos, sl1, 0.0))

    cmax = conf_ref[0, 0]
    for cc in range(1, C):
        cmax = jnp.maximum(cmax, conf_ref[0, cc])
    ssum = jnp.zeros((_G, D), jnp.float32)
    picked = jnp.zeros((_G, D), jnp.float32)
    for cc in range(C):
        plane = conf_ref[0, cc]
        ssum = ssum + jnp.exp(plane - cmax)
        picked = jnp.where(conf_t == cc, plane, picked)
    loss_c = cmax + jnp.log(ssum) - picked

    masked = jnp.where(pos, 0.0, loss_c)
    bits = jax.lax.bitcast_convert_type(masked, jnp.int32)
    k = jnp.minimum(num_pos.astype(jnp.int32) * _NEGPOS_RATIO, D)

    def body(_, lohi):
        lo, hi = lohi
        mid = lo + (hi - lo + 1) // 2
        cnt = jnp.sum((bits >= mid).astype(jnp.int32), axis=1, keepdims=True)
        ok = cnt >= k
        return jnp.where(ok, mid, lo), jnp.where(ok, hi, mid - 1)

    lo0 = jnp.zeros((_G, 1), jnp.int32)
    hi0 = jnp.full((_G, 1), 0x7F7FFFFF, jnp.int32)
    lo, _ = jax.lax.fori_loop(0, 31, body, (lo0, hi0))
    vk = jax.lax.bitcast_convert_type(lo, jnp.float32)
    gt = masked > vk
    cnt_gt = jnp.sum(gt.astype(jnp.float32), axis=1, keepdims=True)
    sum_gt = jnp.sum(jnp.where(gt, masked, 0.0), axis=1, keepdims=True)
    kf = k.astype(jnp.float32)
    topk = jnp.where(k > 0, sum_gt + (kf - cnt_gt) * vk, 0.0)
    loss_c_tot = jnp.sum(jnp.where(pos, loss_c, 0.0)) + jnp.sum(topk)

    @pl.when(b == 0)
    def _init():
        out_ref[0, 0] = loss_l
        out_ref[0, 1] = loss_c_tot
        out_ref[0, 2] = jnp.sum(num_pos)

    @pl.when(b != 0)
    def _acc():
        out_ref[0, 0] += loss_l
        out_ref[0, 1] += loss_c_tot
        out_ref[0, 2] += jnp.sum(num_pos)


@jax.jit
def kernel(loc, conf, dbox_list, targets):
    B, D, C = conf.shape
    nb = B // _G
    loc_r = loc.reshape(nb, _G, D, 4).transpose(0, 3, 1, 2)
    conf_r = conf.reshape(nb, _G, D, C).transpose(0, 3, 1, 2)
    tgt_r = targets.reshape(nb, _G, _NOBJ * 5)
    pf = jnp.concatenate(
        [dbox_list[:, :2] - dbox_list[:, 2:] / 2.0,
         dbox_list[:, :2] + dbox_list[:, 2:] / 2.0], axis=1).T
    dbox_r = dbox_list.T

    out = pl.pallas_call(
        _loss_kernel,
        grid=(nb,),
        in_specs=[
            pl.BlockSpec((1, _G, _NOBJ * 5), lambda b: (b, 0, 0)),
            pl.BlockSpec((1, 4, _G, D), lambda b: (b, 0, 0, 0)),
            pl.BlockSpec((1, C, _G, D), lambda b: (b, 0, 0, 0)),
            pl.BlockSpec((4, D), lambda b: (0, 0)),
            pl.BlockSpec((4, D), lambda b: (0, 0)),
        ],
        out_specs=pl.BlockSpec((1, 3), lambda b: (0, 0), memory_space=pltpu.SMEM),
        out_shape=jax.ShapeDtypeStruct((1, 3), jnp.float32),
    )(tgt_r, loc_r, conf_r, pf, dbox_r)

    N = out[0, 2]
    return (out[0, 0] / N, out[0, 1] / N)
